# baseline (speedup 1.0000x reference)
import functools

import jax
import jax.numpy as jnp
from jax import lax
from jax.experimental import pallas as pl
from jax.experimental.pallas import tpu as pltpu
from jax.experimental.pallas import tpu_sc as plsc

B = 32
N = 20000
CIN = 10
COUT = 9
SCALE = 1.0 / 640.0

NW = 32
TCOLS = 157
FULLCOLS = TCOLS - 1
ROUNDS = 5

IN_PLANES = [0, 1, 2, 3]
NBUF = len(IN_PLANES)
COPY_PAIRS = [(4, 6), (5, 7), (6, 8), (7, 9), (8, 4)]

_mesh = plsc.VectorSubcoreMesh(core_axis_name="c", subcore_axis_name="s")


SETS = 3
LASTW = FULLCOLS - (ROUNDS - 1) * NW


@functools.partial(
    pl.kernel,
    mesh=_mesh,
    out_type=jax.ShapeDtypeStruct((COUT, B, N), jnp.float32),
    compiler_params=pltpu.CompilerParams(needs_layout_passes=False),
    scratch_types=(
        [pltpu.VMEM((16, 256), jnp.float32) for _ in range(SETS * NBUF)]
        + [pltpu.SemaphoreType.DMA for _ in range(2 * SETS + 1)]
    ),
)
def _adapter(pred_hbm, out_hbm, *refs):
    bufs = [dict(zip(IN_PLANES, refs[s * NBUF:(s + 1) * NBUF]))
            for s in range(SETS)]
    sem_i = refs[SETS * NBUF:SETS * NBUF + SETS]
    sem_o = refs[SETS * NBUF + SETS:SETS * NBUF + 2 * SETS]
    sem_hh = refs[SETS * NBUF + 2 * SETS]

    cid = lax.axis_index("c")
    sid = lax.axis_index("s")
    wid = sid * 2 + cid

    def compute(buf):
        @plsc.parallel_loop(0, 16 * 16, unroll=2)
        def body(i):
            r = i // 16
            j = (i % 16) * 16
            x1 = buf[0][r, pl.ds(j, 16)]
            y1 = buf[1][r, pl.ds(j, 16)]
            x2 = buf[2][r, pl.ds(j, 16)]
            y2 = buf[3][r, pl.ds(j, 16)]
            buf[2][r, pl.ds(j, 16)] = (x2 - x1) * SCALE
            buf[3][r, pl.ds(j, 16)] = (y2 - y1) * SCALE
            buf[0][r, pl.ds(j, 16)] = x1 * SCALE
            buf[1][r, pl.ds(j, 16)] = y1 * SCALE

    def rowcol_of(k):
        u = wid + NW * k
        return (u % 2) * 16, (u // 2) * 256

    def in_copies(k):
        s = k % SETS
        row, col = rowcol_of(k)
        return [pltpu.make_async_copy(
            pred_hbm.at[c, pl.ds(row, 16), pl.ds(col, 256)],
            bufs[s][c], sem_i[s])
            for c in IN_PLANES]

    def out_copies(k):
        s = k % SETS
        row, col = rowcol_of(k)
        return [pltpu.make_async_copy(
            bufs[s][o],
            out_hbm.at[o, pl.ds(row, 16), pl.ds(col, 256)], sem_o[s])
            for o in range(NBUF)]

    def hh_copies(k):
        row, col = rowcol_of(k)
        return [pltpu.make_async_copy(
            pred_hbm.at[c, pl.ds(row, 16), pl.ds(col, 256)],
            out_hbm.at[o, pl.ds(row, 16), pl.ds(col, 256)], sem_hh)
            for o, c in COPY_PAIRS]

    def start(cps):
        for cp in cps:
            cp.start()

    def wait(cps):
        for cp in cps:
            cp.wait()

    start(in_copies(0))
    start(in_copies(1))

    def do_round(k):
        start(hh_copies(k))
        wait(in_copies(k))
        compute(bufs[k % SETS])
        start(out_copies(k))

    for k in range(ROUNDS - 1):
        do_round(k)
        nk = k + 2
        if nk <= ROUNDS - 1:
            pk = nk - SETS
            if pk >= 0:
                wait(out_copies(pk))
            if nk < ROUNDS - 1:
                start(in_copies(nk))
            else:
                @pl.when(wid < LASTW)
                def _issue_last():
                    start(in_copies(nk))

    @pl.when(wid < LASTW)
    def _last_round():
        do_round(ROUNDS - 1)

    wait(out_copies(ROUNDS - 3))
    wait(out_copies(ROUNDS - 2))
    for k in range(ROUNDS - 1):
        wait(hh_copies(k))

    @pl.when(wid < LASTW)
    def _drain_last():
        wait(out_copies(ROUNDS - 1))
        wait(hh_copies(ROUNDS - 1))


def _tail_body(x_ref, alias_ref, o_ref):
    x = x_ref[...]
    bb = x[0:4]
    wh = bb[2:4] - bb[0:2]
    o_ref[...] = jnp.concatenate(
        [bb[0:2] * SCALE, wh * SCALE, x[6:10], x[4:5]], axis=0)


_tail_call = pl.pallas_call(
    _tail_body,
    out_shape=jax.ShapeDtypeStruct((COUT, B, N), jnp.float32),
    grid=(1,),
    in_specs=[
        pl.BlockSpec((CIN, B, 128), lambda i: (0, 0, FULLCOLS)),
        pl.BlockSpec(memory_space=pl.ANY),
    ],
    out_specs=pl.BlockSpec((COUT, B, 128), lambda i: (0, 0, FULLCOLS)),
    input_output_aliases={1: 0},
)


def kernel(predictions):
    planar = jnp.transpose(predictions, (2, 0, 1))
    main = _adapter(planar)
    full = _tail_call(planar, main)
    return jnp.transpose(full, (1, 2, 0))


# speedup vs baseline: 10.5978x; 10.5978x over previous
import functools

import jax
import jax.numpy as jnp
from jax import lax
from jax.experimental import pallas as pl
from jax.experimental.pallas import tpu as pltpu
from jax.experimental.pallas import tpu_sc as plsc

B = 32
N = 20000
CIN = 10
COUT = 9
SCALE = 1.0 / 640.0

NW = 32
TCOLS = 157
FULLCOLS = TCOLS - 1
ROUNDS = 5

IN_PLANES = [0, 1, 2, 3, 4, 6, 7, 8, 9]
OUT_SRC = [0, 1, 2, 3, 6, 7, 8, 9, 4]
NBUF = len(IN_PLANES)

_mesh = plsc.VectorSubcoreMesh(core_axis_name="c", subcore_axis_name="s")


SETS = 3
LASTW = FULLCOLS - (ROUNDS - 1) * NW


@functools.partial(
    pl.kernel,
    mesh=_mesh,
    out_type=jax.ShapeDtypeStruct((COUT, B, N), jnp.float32),
    compiler_params=pltpu.CompilerParams(needs_layout_passes=False),
    scratch_types=(
        [pltpu.VMEM((16, 256), jnp.float32) for _ in range(SETS * NBUF)]
        + [pltpu.SemaphoreType.DMA for _ in range(2 * SETS)]
    ),
)
def _adapter(pred_hbm, out_hbm, *refs):
    bufs = [dict(zip(IN_PLANES, refs[s * NBUF:(s + 1) * NBUF]))
            for s in range(SETS)]
    sem_i = refs[SETS * NBUF:SETS * NBUF + SETS]
    sem_o = refs[SETS * NBUF + SETS:SETS * NBUF + 2 * SETS]

    cid = lax.axis_index("c")
    sid = lax.axis_index("s")
    wid = sid * 2 + cid

    def compute(buf):
        @plsc.parallel_loop(0, 16 * 16, unroll=2)
        def body(i):
            r = i // 16
            j = (i % 16) * 16
            x1 = buf[0][r, pl.ds(j, 16)]
            y1 = buf[1][r, pl.ds(j, 16)]
            x2 = buf[2][r, pl.ds(j, 16)]
            y2 = buf[3][r, pl.ds(j, 16)]
            buf[2][r, pl.ds(j, 16)] = (x2 - x1) * SCALE
            buf[3][r, pl.ds(j, 16)] = (y2 - y1) * SCALE
            buf[0][r, pl.ds(j, 16)] = x1 * SCALE
            buf[1][r, pl.ds(j, 16)] = y1 * SCALE

    def rowcol_of(k):
        u = wid + NW * k
        return (u % 2) * 16, (u // 2) * 256

    IN_ORDER = [4, 6, 7, 8, 9, 0, 1, 2, 3]
    COPY_OUT = [(4, 6), (5, 7), (6, 8), (7, 9), (8, 4)]
    BBOX_OUT = [(0, 0), (1, 1), (2, 2), (3, 3)]

    def in_copies(k):
        s = k % SETS
        row, col = rowcol_of(k)
        return [pltpu.make_async_copy(
            pred_hbm.at[c, pl.ds(row, 16), pl.ds(col, 256)],
            bufs[s][c], sem_i[s])
            for c in IN_ORDER]

    def out_copies(k, pairs=COPY_OUT + BBOX_OUT):
        s = k % SETS
        row, col = rowcol_of(k)
        return [pltpu.make_async_copy(
            bufs[s][src],
            out_hbm.at[o, pl.ds(row, 16), pl.ds(col, 256)], sem_o[s])
            for o, src in pairs]

    def start(cps):
        for cp in cps:
            cp.start()

    def wait(cps):
        for cp in cps:
            cp.wait()

    start(in_copies(0))
    start(in_copies(1))

    def do_round(k):
        ins = in_copies(k)
        wait(ins[:5])
        start(out_copies(k, COPY_OUT))
        wait(ins[5:])
        compute(bufs[k % SETS])
        start(out_copies(k, BBOX_OUT))

    for k in range(ROUNDS - 1):
        do_round(k)
        nk = k + 2
        if nk <= ROUNDS - 1:
            pk = nk - SETS
            if pk >= 0:
                wait(out_copies(pk))
            if nk < ROUNDS - 1:
                start(in_copies(nk))
            else:
                @pl.when(wid < LASTW)
                def _issue_last():
                    start(in_copies(nk))

    @pl.when(wid < LASTW)
    def _last_round():
        do_round(ROUNDS - 1)

    wait(out_copies(ROUNDS - 3))
    wait(out_copies(ROUNDS - 2))

    @pl.when(wid < LASTW)
    def _drain_last():
        wait(out_copies(ROUNDS - 1))


def _tail_body(x_ref, alias_ref, o_ref):
    x = x_ref[...]
    bb = x[0:4]
    wh = bb[2:4] - bb[0:2]
    o_ref[...] = jnp.concatenate(
        [bb[0:2] * SCALE, wh * SCALE, x[6:10], x[4:5]], axis=0)


_tail_call = pl.pallas_call(
    _tail_body,
    out_shape=jax.ShapeDtypeStruct((COUT, B, N), jnp.float32),
    grid=(1,),
    in_specs=[
        pl.BlockSpec((CIN, B, 128), lambda i: (0, 0, FULLCOLS)),
        pl.BlockSpec(memory_space=pl.ANY),
    ],
    out_specs=pl.BlockSpec((COUT, B, 128), lambda i: (0, 0, FULLCOLS)),
    input_output_aliases={1: 0},
)


def kernel(predictions):
    planar = jnp.transpose(predictions, (2, 0, 1))
    main = _adapter(planar)
    full = _tail_call(planar, main)
    return jnp.transpose(full, (1, 2, 0))
